# baseline (speedup 1.0000x reference)
import functools

import jax
import jax.numpy as jnp
from jax import lax
from jax.experimental import pallas as pl
from jax.experimental.pallas import tpu as pltpu
from jax.experimental.pallas import tpu_sc as plsc

B = 128
VOCAB = 100000
TOPK = 5
L = 16

SCROWS = 32
RSPLIT = B - SCROWS
CHUNKS = VOCAB // L
UNROLL = 10
OUTER = CHUNKS // UNROLL

RB = 8
NRB = RSPLIT // RB


def _sc_scan_body(yp_hbm, y_hbm, cnt_hbm, yv, row_v, cv, sem):
    c = lax.axis_index("c")
    s = lax.axis_index("s")
    wid = s * 2 + c
    b = RSPLIT + wid
    pltpu.sync_copy(y_hbm, yv)
    pltpu.sync_copy(yp_hbm.at[b], row_v)
    iota = lax.iota(jnp.int32, L)
    bsplat = jnp.zeros((L,), jnp.int32) + b
    yb = plsc.load_gather(yv, [bsplat])
    t = plsc.load_gather(row_v, [yb])

    def step(o, acc):
        base = o * (L * UNROLL)
        for u in range(UNROLL):
            cbase = base + u * L
            col = iota + cbase
            v = plsc.load_gather(row_v, [col])
            m = (v > t) | ((v == t) & (col < yb))
            acc = acc + plsc.all_reduce_population_count(m)
        return acc

    acc = lax.fori_loop(0, OUTER, step, jnp.zeros((L,), jnp.int32))
    cv[...] = acc
    pltpu.sync_copy(cv, cnt_hbm.at[wid])


@functools.cache
def _sc_scan():
    return pl.kernel(
        _sc_scan_body,
        out_type=jax.ShapeDtypeStruct((SCROWS, L), jnp.int32),
        mesh=plsc.VectorSubcoreMesh(core_axis_name="c", subcore_axis_name="s"),
        compiler_params=pltpu.CompilerParams(needs_layout_passes=False),
        scratch_types=[
            pltpu.VMEM((B,), jnp.int32),
            pltpu.VMEM((VOCAB,), jnp.float32),
            pltpu.VMEM((L,), jnp.int32),
            pltpu.SemaphoreType.DMA,
        ],
    )


def _tc_scan_body(y_ref, x_ref, out_ref, acc_ref):
    i = pl.program_id(0)
    yy = y_ref[...]
    vals = x_ref[...]
    col = lax.broadcasted_iota(jnp.int32, (RB, VOCAB), 1)
    t = jnp.sum(
        jnp.where(col == yy, vals, 0.0), axis=1, keepdims=True
    )
    m = (vals > t) | ((vals == t) & (col < yy))
    acc_ref[pl.ds(i * RB, RB), :] = jnp.sum(
        m.astype(jnp.int32), axis=1, keepdims=True
    )

    @pl.when(i == NRB - 1)
    def _():
        out_ref[...] = jnp.sum(
            (acc_ref[...] < TOPK).astype(jnp.int32), axis=(0, 1), keepdims=True
        )


def _tc_scan(y_pred, y):
    return pl.pallas_call(
        _tc_scan_body,
        grid=(NRB,),
        in_specs=[
            pl.BlockSpec((RB, 1), lambda i: (i, 0)),
            pl.BlockSpec((RB, VOCAB), lambda i: (i, 0)),
        ],
        out_specs=pl.BlockSpec((1, 1), lambda i: (0, 0)),
        out_shape=jax.ShapeDtypeStruct((1, 1), jnp.int32),
        scratch_shapes=[
            pltpu.VMEM((RSPLIT, 1), jnp.int32),
        ],
    )(y[:RSPLIT].reshape(RSPLIT, 1), y_pred)


def _combine_body(tc_ref, cnt_ref, out_ref):
    sc_hits = jnp.sum(
        (cnt_ref[...][:, :1] < TOPK).astype(jnp.int32), axis=(0, 1), keepdims=True
    )
    out_ref[...] = tc_ref[...] + sc_hits


def _combine(tc_hits, cnt):
    return pl.pallas_call(
        _combine_body,
        out_shape=jax.ShapeDtypeStruct((1, 1), jnp.int32),
    )(tc_hits, cnt)


def kernel(y_pred, y):
    y32 = y.astype(jnp.int32)
    cnt = _sc_scan()(y_pred, y32)
    tc_hits = _tc_scan(y_pred, y32)
    return _combine(tc_hits, cnt)[0, 0]
